# Initial kernel scaffold; baseline (speedup 1.0000x reference)
import jax
import jax.numpy as jnp
from jax.experimental import pallas as pl


def kernel(x, indices, W_edge, b_edge, W1, b1, W2, b2, Wf, bf):
    raise NotImplementedError("write your pallas kernel here")


# speedup vs baseline: 2294.8660x; 2294.8660x over previous
import functools

import jax
import jax.numpy as jnp
from jax import lax
from jax.experimental import pallas as pl
from jax.experimental.pallas import tpu as pltpu
from jax.experimental.pallas import tpu_sc as plsc

B, N, K = 8, 4096, 20
C = 64
CG = 4
CPG = C // CG
NCHUNK = 4
CHUNK = N // NCHUNK
H0, H1, H2 = 64, 64, 128
WDIM = 512


def _lrelu(v):
    return jnp.where(v > 0, v, v * jnp.float32(0.2))


def _make_gmax(with_epilogue):
    mesh = plsc.VectorSubcoreMesh(core_axis_name="c", subcore_axis_name="s")

    scratch = [
        pltpu.VMEM((CPG * N,), jnp.float32),
        pltpu.VMEM((K, CHUNK), jnp.int32),
        pltpu.VMEM((CPG, CHUNK), jnp.float32),
    ]
    if with_epilogue:
        scratch.append(pltpu.VMEM((CPG, CHUNK), jnp.float32))

    def body(*refs):
        if with_epilogue:
            src_hbm, idx_hbm, cep_hbm, out_hbm, slab, idxb, outb, cepb = refs
        else:
            src_hbm, idx_hbm, out_hbm, slab, idxb, outb = refs
            cep_hbm = cepb = None

        wid = lax.axis_index("s") * 2 + lax.axis_index("c")
        b = wid // CG
        cg = wid % CG

        pltpu.sync_copy(src_hbm.at[b, pl.ds(cg * (CPG * N), CPG * N)], slab)

        neg_inf = jnp.full((16,), -jnp.inf, dtype=jnp.float32)

        for ch in range(NCHUNK):
            pltpu.sync_copy(idx_hbm.at[b, :, pl.ds(ch * CHUNK, CHUNK)], idxb)
            if with_epilogue:
                pltpu.sync_copy(
                    cep_hbm.at[b, pl.ds(cg * CPG, CPG),
                               pl.ds(ch * CHUNK, CHUNK)], cepb)

            def blk(i, carry):
                base = i * 16
                idxvs = [idxb[k, pl.ds(base, 16)] for k in range(K)]

                def cgrp(g, carry2):
                    accs = [neg_inf] * 4
                    for k in range(K):
                        for j in range(4):
                            off = (g * 4 + j) * N
                            v = plsc.load_gather(slab, [idxvs[k] + off])
                            accs[j] = jnp.maximum(accs[j], v)
                    for j in range(4):
                        r = accs[j]
                        if with_epilogue:
                            r = _lrelu(r + cepb[g * 4 + j, pl.ds(base, 16)])
                        outb[g * 4 + j, pl.ds(base, 16)] = r
                    return carry2

                lax.fori_loop(0, CPG // 4, cgrp, 0)
                return carry

            lax.fori_loop(0, CHUNK // 16, blk, 0)
            pltpu.sync_copy(
                outb,
                out_hbm.at[b, pl.ds(cg * CPG, CPG), pl.ds(ch * CHUNK, CHUNK)])

    out_type = jax.ShapeDtypeStruct((B, C, N), jnp.float32)
    return pl.kernel(body, out_type=out_type, mesh=mesh,
                     scratch_types=scratch,
                     compiler_params=pltpu.CompilerParams(
                         needs_layout_passes=False))


_gmax_lrelu = _make_gmax(True)
_gmax_plain = _make_gmax(False)


def _tca_body(xt_ref, w_ref, be_ref, y0_ref, c0_ref):
    xb = xt_ref[0]
    wa = w_ref[:, 0:3]
    wb = w_ref[:, 3:6]
    y0 = (wa[:, 0:1] * xb[0:1, :] + wa[:, 1:2] * xb[1:2, :]
          + wa[:, 2:3] * xb[2:3, :])
    wd = wb - wa
    c0 = (wd[:, 0:1] * xb[0:1, :] + wd[:, 1:2] * xb[1:2, :]
          + wd[:, 2:3] * xb[2:3, :]) + be_ref[...]
    y0_ref[0] = y0
    c0_ref[0] = c0


def _tca(xt, W_edge, b_edge):
    return pl.pallas_call(
        _tca_body,
        grid=(B,),
        in_specs=[
            pl.BlockSpec((1, 3, N), lambda b: (b, 0, 0)),
            pl.BlockSpec((H0, 6), lambda b: (0, 0)),
            pl.BlockSpec((H0, 1), lambda b: (0, 0)),
        ],
        out_specs=[
            pl.BlockSpec((1, H0, N), lambda b: (b, 0, 0)),
            pl.BlockSpec((1, H0, N), lambda b: (b, 0, 0)),
        ],
        out_shape=[
            jax.ShapeDtypeStruct((B, H0, N), jnp.float32),
            jax.ShapeDtypeStruct((B, H0, N), jnp.float32),
        ],
    )(xt, W_edge, b_edge)


def _tcb_body(g1_ref, w1_ref, b1_ref, out_ref):
    h = jnp.dot(w1_ref[...], g1_ref[0],
                preferred_element_type=jnp.float32) + b1_ref[...]
    out_ref[0] = _lrelu(h)


def _tcb(g1, W1, b1):
    return pl.pallas_call(
        _tcb_body,
        grid=(B,),
        in_specs=[
            pl.BlockSpec((1, H0, N), lambda b: (b, 0, 0)),
            pl.BlockSpec((H1, H0), lambda b: (0, 0)),
            pl.BlockSpec((H1, 1), lambda b: (0, 0)),
        ],
        out_specs=pl.BlockSpec((1, H1, N), lambda b: (b, 0, 0)),
        out_shape=jax.ShapeDtypeStruct((B, H1, N), jnp.float32),
    )(g1, W1, b1)


def _tcc_body(g2_ref, h0_ref, h1_ref, w2_ref, b2_ref, wf0_ref, wf1_ref,
              wf2_ref, bf_ref, out_ref):
    h2 = _lrelu(jnp.dot(w2_ref[...], g2_ref[0],
                        preferred_element_type=jnp.float32) + b2_ref[...])
    p = (jnp.dot(wf0_ref[...], h0_ref[0], preferred_element_type=jnp.float32)
         + jnp.dot(wf1_ref[...], h1_ref[0],
                   preferred_element_type=jnp.float32)
         + jnp.dot(wf2_ref[...], h2, preferred_element_type=jnp.float32))
    out_ref[...] = jnp.max(p, axis=1)[None, None, :] + bf_ref[...][None]


def _tcc(g2, h0, h1, W2, b2, Wf, bf):
    return pl.pallas_call(
        _tcc_body,
        grid=(B,),
        in_specs=[
            pl.BlockSpec((1, H1, N), lambda b: (b, 0, 0)),
            pl.BlockSpec((1, H0, N), lambda b: (b, 0, 0)),
            pl.BlockSpec((1, H1, N), lambda b: (b, 0, 0)),
            pl.BlockSpec((H2, H1), lambda b: (0, 0)),
            pl.BlockSpec((H2, 1), lambda b: (0, 0)),
            pl.BlockSpec((WDIM, H0), lambda b: (0, 0)),
            pl.BlockSpec((WDIM, H1), lambda b: (0, 0)),
            pl.BlockSpec((WDIM, H2), lambda b: (0, 0)),
            pl.BlockSpec((1, WDIM), lambda b: (0, 0)),
        ],
        out_specs=pl.BlockSpec((1, 1, WDIM), lambda b: (b, 0, 0)),
        out_shape=jax.ShapeDtypeStruct((B, 1, WDIM), jnp.float32),
    )(g2, h0, h1, W2, b2, Wf[:, :H0], Wf[:, H0:H0 + H1], Wf[:, H0 + H1:], bf)


def kernel(x, indices, W_edge, b_edge, W1, b1, W2, b2, Wf, bf):
    idx_t = jnp.transpose(indices.astype(jnp.int32), (0, 2, 1))
    xt = jnp.transpose(x, (0, 2, 1))

    y0, c0 = _tca(xt, W_edge, b_edge.reshape(H0, 1))
    h0 = _gmax_lrelu(y0.reshape(B, C * N), idx_t, c0)
    g1 = _gmax_plain(h0.reshape(B, C * N), idx_t)
    h1 = _tcb(g1, W1, b1.reshape(H1, 1))
    g2 = _gmax_plain(h1.reshape(B, C * N), idx_t)
    out = _tcc(g2, h0, h1, W2, b2.reshape(H2, 1), Wf, bf.reshape(1, WDIM))
    return out.reshape(B, WDIM)
